# baseline (speedup 1.0000x reference)
import functools
import jax
import jax.numpy as jnp
from jax import lax
from jax.experimental import pallas as pl
from jax.experimental.pallas import tpu as pltpu
from jax.experimental.pallas import tpu_sc as plsc

_CH = 32768


def kernel(x, pos_table):
    b, s, d = x.shape
    n = b * s * d
    npos = s * d
    info = plsc.get_sparse_core_info()
    nw = info.num_cores * info.num_subcores
    span = n // nw
    nchunks = span // _CH
    wrap = npos // span

    mesh = plsc.VectorSubcoreMesh(core_axis_name="c", subcore_axis_name="s")

    @functools.partial(
        pl.kernel,
        out_type=jax.ShapeDtypeStruct((n,), jnp.float32),
        mesh=mesh,
        scratch_types=[
            pltpu.VMEM((_CH,), jnp.float32),
            pltpu.VMEM((_CH,), jnp.float32),
        ],
    )
    def sc_add(x_hbm, pos_hbm, out_hbm, xv, pv):
        wid = lax.axis_index("s") * info.num_cores + lax.axis_index("c")
        base = wid * span
        pbase = (wid % wrap) * span
        for k in range(nchunks):
            off = k * _CH
            pltpu.sync_copy(x_hbm.at[pl.ds(base + off, _CH)], xv)
            pltpu.sync_copy(pos_hbm.at[pl.ds(pbase + off, _CH)], pv)

            def body(i, _):
                sl = pl.ds(i * 16, 16)
                xv[sl] = xv[sl] + pv[sl]
                return 0

            lax.fori_loop(0, _CH // 16, body, 0)
            pltpu.sync_copy(xv, out_hbm.at[pl.ds(base + off, _CH)])

    out = sc_add(x.reshape(n), pos_table.reshape(npos))
    return out.reshape(b, s, d)


# speedup vs baseline: 2.5918x; 2.5918x over previous
import functools
import jax
import jax.numpy as jnp
from jax import lax
from jax.experimental import pallas as pl
from jax.experimental.pallas import tpu as pltpu
from jax.experimental.pallas import tpu_sc as plsc

_SC_ROWS = 1024
_TC_BLOCK = 1024


def _tc_add(x_ref, pos_ref, o_ref):
    o_ref[...] = x_ref[...] + pos_ref[...]


def kernel(x, pos_table):
    b, s, d = x.shape
    rows = b * s
    xf = x.reshape(rows, d)

    info = plsc.get_sparse_core_info()
    nw = info.num_cores * info.num_subcores
    span = _SC_ROWS * d // nw
    mesh = plsc.VectorSubcoreMesh(core_axis_name="c", subcore_axis_name="s")

    @functools.partial(
        pl.kernel,
        out_type=jax.ShapeDtypeStruct((_SC_ROWS * d,), jnp.float32),
        mesh=mesh,
        scratch_types=[
            pltpu.VMEM((span,), jnp.float32),
            pltpu.VMEM((span,), jnp.float32),
        ],
    )
    def sc_add(x_hbm, pos_hbm, out_hbm, xv, pv):
        wid = lax.axis_index("s") * info.num_cores + lax.axis_index("c")
        base = wid * span
        pltpu.sync_copy(x_hbm.at[pl.ds(base, span)], xv)
        pltpu.sync_copy(pos_hbm.at[pl.ds(base, span)], pv)

        def body(i, _):
            sl = pl.ds(i * 16, 16)
            xv[sl] = xv[sl] + pv[sl]
            return 0

        lax.fori_loop(0, span // 16, body, 0)
        pltpu.sync_copy(xv, out_hbm.at[pl.ds(base, span)])

    sc_out = sc_add(
        xf[:_SC_ROWS].reshape(-1), pos_table[:_SC_ROWS].reshape(-1)
    ).reshape(_SC_ROWS, d)

    tc_rows = rows - _SC_ROWS
    nblk = tc_rows // _TC_BLOCK
    pos_period = s // _TC_BLOCK
    off = _SC_ROWS // _TC_BLOCK

    tc_out = pl.pallas_call(
        _tc_add,
        grid=(nblk,),
        in_specs=[
            pl.BlockSpec((_TC_BLOCK, d), lambda j: (j + off, 0)),
            pl.BlockSpec((_TC_BLOCK, d), lambda j: ((j + off) % pos_period, 0)),
        ],
        out_specs=pl.BlockSpec((_TC_BLOCK, d), lambda j: (j, 0)),
        out_shape=jax.ShapeDtypeStruct((tc_rows, d), jnp.float32),
        compiler_params=pltpu.CompilerParams(
            dimension_semantics=("arbitrary",),
        ),
    )(xf, pos_table)

    return jnp.concatenate([sc_out, tc_out], axis=0).reshape(b, s, d)


# speedup vs baseline: 9.0212x; 3.4807x over previous
import jax
import jax.numpy as jnp
from jax.experimental import pallas as pl
from jax.experimental.pallas import tpu as pltpu

def _add_kernel(x_ref, pos_ref, o_ref):
    o_ref[...] = x_ref[...] + pos_ref[...]


def kernel(x, pos_table):
    b, s, d = x.shape
    return pl.pallas_call(
        _add_kernel,
        grid=(b,),
        in_specs=[
            pl.BlockSpec((1, s, d), lambda j: (j, 0, 0)),
            pl.BlockSpec((s, d), lambda j: (0, 0)),
        ],
        out_specs=pl.BlockSpec((1, s, d), lambda j: (j, 0, 0)),
        out_shape=jax.ShapeDtypeStruct((b, s, d), x.dtype),
        compiler_params=pltpu.CompilerParams(
            dimension_semantics=("arbitrary",),
        ),
    )(x, pos_table)
